# baseline (speedup 1.0000x reference)
import jax
import jax.numpy as jnp
from jax.experimental import pallas as pl
from jax.experimental.pallas import tpu as pltpu

_EPS = 1e-12


def _half_vlad(x, wt):
    ssq = jnp.sum(x * x, axis=1, keepdims=True)
    rinv = jax.lax.rsqrt(jnp.maximum(ssq, _EPS * _EPS))
    logits = jnp.dot(x, wt, preferred_element_type=jnp.float32) * rinv
    m = jnp.max(logits, axis=1, keepdims=True)
    e = jnp.exp(logits - m)
    a = e / jnp.sum(e, axis=1, keepdims=True)
    a2 = a * rinv
    vlad = jax.lax.dot_general(
        a2, x, (((0,), (0,)), ((), ())), preferred_element_type=jnp.float32
    )
    asum = jnp.sum(a, axis=0, keepdims=True)
    return vlad, asum


def _netvlad_block(x1_ref, x2_ref, x3_ref, x4_ref, wt_ref, c_ref, o_ref):
    wt = wt_ref[...]
    c = c_ref[...]

    v1, s1 = _half_vlad(x1_ref[0, 0], wt)
    v2, s2 = _half_vlad(x2_ref[0, 0], wt)
    v3, s3 = _half_vlad(x3_ref[0, 0], wt)
    v4, s4 = _half_vlad(x4_ref[0, 0], wt)
    vlad = ((v1 + v2) + (v3 + v4)) - ((s1 + s2) + (s3 + s4)).T * c

    rn = jnp.sqrt(jnp.sum(vlad * vlad, axis=1, keepdims=True))
    vlad = vlad / jnp.maximum(rn, _EPS)
    g = jnp.sqrt(jnp.sum(vlad * vlad, keepdims=True))
    o_ref[0] = vlad / jnp.maximum(g, _EPS)


@jax.jit
def kernel(x, conv_w, centroids):
    B, D, H, W = x.shape
    K = centroids.shape[0]
    N = H * W
    Nh = N // 4
    xt = jnp.transpose(x, (0, 2, 3, 1)).reshape(B, 4, Nh, D)
    out = pl.pallas_call(
        _netvlad_block,
        grid=(B,),
        in_specs=[
            pl.BlockSpec((1, 1, Nh, D), lambda b: (b, 0, 0, 0)),
            pl.BlockSpec((1, 1, Nh, D), lambda b: (b, 1, 0, 0)),
            pl.BlockSpec((1, 1, Nh, D), lambda b: (b, 2, 0, 0)),
            pl.BlockSpec((1, 1, Nh, D), lambda b: (b, 3, 0, 0)),
            pl.BlockSpec((D, K), lambda b: (0, 0)),
            pl.BlockSpec((K, D), lambda b: (0, 0)),
        ],
        out_specs=pl.BlockSpec((1, K, D), lambda b: (b, 0, 0)),
        out_shape=jax.ShapeDtypeStruct((B, K, D), jnp.float32),
        compiler_params=pltpu.CompilerParams(
            dimension_semantics=("parallel",),
        ),
    )(xt, xt, xt, xt, conv_w.T, centroids)
    return out.reshape(B, K * D)


# speedup vs baseline: 1.7217x; 1.7217x over previous
import jax
import jax.numpy as jnp
from jax.experimental import pallas as pl
from jax.experimental.pallas import tpu as pltpu

_EPS = 1e-12
_N_STREAMS = 6


def _part_vlad(x, wt):
    ssq = jnp.sum(x * x, axis=1, keepdims=True)
    rinv = jax.lax.rsqrt(jnp.maximum(ssq, _EPS * _EPS))
    logits = jnp.dot(x, wt, preferred_element_type=jnp.float32) * rinv
    m = jnp.max(logits, axis=1, keepdims=True)
    e = jnp.exp(logits - m)
    a = e / jnp.sum(e, axis=1, keepdims=True)
    a2 = a * rinv
    vlad = jax.lax.dot_general(
        a2, x, (((0,), (0,)), ((), ())), preferred_element_type=jnp.float32
    )
    asum = jnp.sum(a, axis=0, keepdims=True)
    return vlad, asum


def _netvlad_block(*refs):
    x_refs = refs[:_N_STREAMS]
    wt_ref, c_ref, o_ref = refs[_N_STREAMS:]
    wt = wt_ref[...]
    c = c_ref[...]

    vlad = None
    asum = None
    for xr in x_refs:
        v, s = _part_vlad(xr[0], wt)
        vlad = v if vlad is None else vlad + v
        asum = s if asum is None else asum + s
    vlad = vlad - asum.T * c

    rn = jnp.sqrt(jnp.sum(vlad * vlad, axis=1, keepdims=True))
    vlad = vlad / jnp.maximum(rn, _EPS)
    g = jnp.sqrt(jnp.sum(vlad * vlad, keepdims=True))
    o_ref[0] = vlad / jnp.maximum(g, _EPS)


@jax.jit
def kernel(x, conv_w, centroids):
    B, D, H, W = x.shape
    K = centroids.shape[0]
    N = H * W
    Ns = N // _N_STREAMS
    xt = jnp.transpose(x, (0, 2, 3, 1)).reshape(B, N, D)
    x_specs = [
        pl.BlockSpec((1, Ns, D), lambda b, q=q: (b, q, 0))
        for q in range(_N_STREAMS)
    ]
    out = pl.pallas_call(
        _netvlad_block,
        grid=(B,),
        in_specs=x_specs + [
            pl.BlockSpec((D, K), lambda b: (0, 0)),
            pl.BlockSpec((K, D), lambda b: (0, 0)),
        ],
        out_specs=pl.BlockSpec((1, K, D), lambda b: (b, 0, 0)),
        out_shape=jax.ShapeDtypeStruct((B, K, D), jnp.float32),
        compiler_params=pltpu.CompilerParams(
            dimension_semantics=("parallel",),
        ),
    )(*([xt] * _N_STREAMS), conv_w.T, centroids)
    return out.reshape(B, K * D)


# speedup vs baseline: 3.1784x; 1.8461x over previous
import jax
import jax.numpy as jnp
from jax.experimental import pallas as pl
from jax.experimental.pallas import tpu as pltpu

_EPS = 1e-12
_G = 4


def _part_vlad(x, wt):
    ssq = jnp.sum(x * x, axis=1, keepdims=True)
    rinv = jax.lax.rsqrt(jnp.maximum(ssq, _EPS * _EPS))
    logits = jnp.dot(x, wt, preferred_element_type=jnp.float32) * rinv
    m = jnp.max(logits, axis=1, keepdims=True)
    e = jnp.exp(logits - m)
    a = e / jnp.sum(e, axis=1, keepdims=True)
    a2 = a * rinv
    vlad = jax.lax.dot_general(
        a2, x, (((0,), (0,)), ((), ())), preferred_element_type=jnp.float32
    )
    asum = jnp.sum(a, axis=0, keepdims=True)
    return vlad, asum


def _finish(vlad, asum, c):
    vlad = vlad - asum.T * c
    rn = jnp.sqrt(jnp.sum(vlad * vlad, axis=1, keepdims=True))
    vlad = vlad / jnp.maximum(rn, _EPS)
    g = jnp.sqrt(jnp.sum(vlad * vlad, keepdims=True))
    return vlad / jnp.maximum(g, _EPS)


def _netvlad_block(x_ref, wt_ref, c_ref, o_ref):
    wt = wt_ref[...]
    c = c_ref[...]
    for g in range(_G):
        v, s = _part_vlad(x_ref[g], wt)
        o_ref[g] = _finish(v, s, c)


@jax.jit
def kernel(x, conv_w, centroids):
    B, D, H, W = x.shape
    K = centroids.shape[0]
    N = H * W
    xt = jnp.transpose(x, (0, 2, 3, 1)).reshape(B, N, D)
    out = pl.pallas_call(
        _netvlad_block,
        grid=(B // _G,),
        in_specs=[
            pl.BlockSpec((_G, N, D), lambda i: (i, 0, 0)),
            pl.BlockSpec((D, K), lambda i: (0, 0)),
            pl.BlockSpec((K, D), lambda i: (0, 0)),
        ],
        out_specs=pl.BlockSpec((_G, K, D), lambda i: (i, 0, 0)),
        out_shape=jax.ShapeDtypeStruct((B, K, D), jnp.float32),
        compiler_params=pltpu.CompilerParams(
            dimension_semantics=("parallel",),
            vmem_limit_bytes=56 * 1024 * 1024,
        ),
    )(xt, conv_w.T, centroids)
    return out.reshape(B, K * D)
